# baseline (speedup 1.0000x reference)
import functools

import jax
import jax.numpy as jnp
from jax import lax
from jax.experimental import pallas as pl
from jax.experimental.pallas import tpu as pltpu
from jax.experimental.pallas import tpu_sc as plsc

BATCH = 4096
MAXLEN = 200
VOCAB_ROWS = 1000000
EMBED = 64
LABELS = 100
LANES = 16
NC = 2
NS = 16
NW = NC * NS
B_PER_W = BATCH // NW
CHUNKS = ((0, 128), (128, 72))


NBUF = 4
UNROLL = 8
NCH = EMBED // LANES


def _pool_body(x_hbm, table_hbm, out_hbm, idx_v, rows_v, pooled_v, sems):
    w = lax.axis_index("s") * NC + lax.axis_index("c")
    base = w * B_PER_W
    pltpu.sync_copy(x_hbm.at[pl.ds(base, B_PER_W)], idx_v)

    def fire(b, k):
        for off, n in CHUNKS:
            pltpu.async_copy(
                table_hbm.at[idx_v.at[b, pl.ds(off, n)]],
                rows_v.at[k].at[pl.ds(off, n)],
                sems.at[k],
            )

    def drain(k):
        pltpu.make_async_copy(
            table_hbm.at[pl.ds(0, MAXLEN)], rows_v.at[k], sems.at[k]
        ).wait()

    def reduce_row(b, k):
        rv = rows_v.at[k]
        zero = jnp.zeros((LANES,), jnp.float32)

        def step(t, accs):
            accs = list(accs)
            for j in range(UNROLL):
                r = t * UNROLL + j
                for c in range(NCH):
                    a = c + NCH * (j % 2)
                    accs[a] = accs[a] + rv[r, pl.ds(c * LANES, LANES)]
            return tuple(accs)

        accs = lax.fori_loop(0, MAXLEN // UNROLL, step, (zero,) * (2 * NCH))
        scale = jnp.float32(1.0 / MAXLEN)
        for c in range(NCH):
            pooled_v[b, pl.ds(c * LANES, LANES)] = (accs[c] + accs[c + NCH]) * scale

    for k in range(NBUF):
        fire(k, k)

    def group(g, carry):
        for k in range(NBUF):
            b = g * NBUF + k
            drain(k)

            @pl.when(g < B_PER_W // NBUF - 1)
            def _():
                fire(b + NBUF, k)

            reduce_row(b, k)
        return carry

    lax.fori_loop(0, B_PER_W // NBUF, group, 0)
    pltpu.sync_copy(pooled_v, out_hbm.at[pl.ds(base, B_PER_W)])


_pool_kernel = functools.partial(
    pl.kernel,
    out_type=jax.ShapeDtypeStruct((BATCH, EMBED), jnp.float32),
    mesh=plsc.VectorSubcoreMesh(core_axis_name="c", subcore_axis_name="s"),
    scratch_types=[
        pltpu.VMEM((B_PER_W, MAXLEN), jnp.int32),
        pltpu.VMEM((NBUF, MAXLEN, EMBED), jnp.float32),
        pltpu.VMEM((B_PER_W, EMBED), jnp.float32),
        pltpu.SemaphoreType.DMA((NBUF,)),
    ],
    compiler_params=pltpu.CompilerParams(use_tc_tiling_on_sc=False),
)(_pool_body)


TBLK = 256
OUTW = 128
NBLK = VOCAB_ROWS // TBLK
TAIL = VOCAB_ROWS - NBLK * TBLK
BPW = -(-NBLK // NW)


def _tr_body(tt_hbm, aux_hbm, out_hbm, in_v, out_v, aux_v, in_sem, out_sem):
    w = lax.axis_index("s") * NC + lax.axis_index("c")

    @pl.when(w == 0)
    def _():
        pltpu.sync_copy(aux_hbm, aux_v)
        pltpu.sync_copy(aux_v, out_hbm.at[pl.ds(NBLK * (TBLK // 2), TAIL // 2)])

    def blk(t):
        return w + NW * t

    def fire_in(t, p):
        pltpu.async_copy(
            tt_hbm.at[:, pl.ds(blk(t) * TBLK, TBLK)],
            in_v.at[p, :, pl.ds(0, TBLK)],
            in_sem.at[p],
        )

    def wait_in(p):
        pltpu.make_async_copy(
            tt_hbm.at[:, pl.ds(0, TBLK)],
            in_v.at[p, :, pl.ds(0, TBLK)],
            in_sem.at[p],
        ).wait()

    def fire_out(t, p):
        pltpu.async_copy(
            out_v.at[p], out_hbm.at[pl.ds(blk(t) * (TBLK // 2), TBLK // 2)],
            out_sem.at[p],
        )

    def wait_out(p):
        pltpu.make_async_copy(
            out_v.at[p], out_hbm.at[pl.ds(0, TBLK // 2)], out_sem.at[p]
        ).wait()

    @pl.when(blk(0) < NBLK)
    def _():
        fire_in(0, 0)

    lanes = lax.iota(jnp.int32, LANES)
    row_vecs = tuple(c * LANES + lanes for c in range(NCH))

    def step(g, carry):
        for k in range(2):
            t = 2 * g + k
            p = k

            @pl.when(blk(t) < NBLK)
            def _():
                wait_in(p)

                @pl.when(blk(t + 1) < NBLK)
                def _():
                    fire_in(t + 1, 1 - p)

                @pl.when(t >= 2)
                def _():
                    wait_out(p)

                in2 = in_v.at[p]
                out2 = out_v.at[p]

                @plsc.parallel_loop(
                    0, TBLK // 2, unroll=8,
                    carry=jnp.zeros((LANES,), jnp.int32),
                )
                def _(v2, col0):
                    for half in range(2):
                        col = col0 + half
                        for c in range(NCH):
                            val = plsc.load_gather(in2, [row_vecs[c], col])
                            out2[v2, pl.ds(half * EMBED + c * LANES, LANES)] = val
                    return col0 + 2

                fire_out(t, p)
        return carry

    lax.fori_loop(0, (BPW + 1) // 2, step, 0)
    wait_out(0)
    wait_out(1)


_tr_kernel = functools.partial(
    pl.kernel,
    out_type=jax.ShapeDtypeStruct((VOCAB_ROWS * EMBED // OUTW, OUTW), jnp.float32),
    mesh=plsc.VectorSubcoreMesh(core_axis_name="c", subcore_axis_name="s"),
    scratch_types=[
        pltpu.VMEM((2, EMBED, TBLK), jnp.float32),
        pltpu.VMEM((2, TBLK // 2, OUTW), jnp.float32),
        pltpu.VMEM((TAIL // 2, OUTW), jnp.float32),
        pltpu.SemaphoreType.DMA((2,)),
        pltpu.SemaphoreType.DMA((2,)),
    ],
    compiler_params=pltpu.CompilerParams(
        use_tc_tiling_on_sc=True, needs_layout_passes=False
    ),
)(_tr_body)


LPAD = 128
BM = 512


def _fc_body(p_ref, w_ref, b_ref, o_ref):
    o_ref[...] = (
        jnp.dot(p_ref[...], w_ref[...], preferred_element_type=jnp.float32)
        + b_ref[0:1, :]
    )


@jax.jit
def kernel(x, table, W, b):
    x = x.astype(jnp.int32)
    tt = table.T
    aux = jnp.reshape(table[NBLK * TBLK:, :], (TAIL // 2, OUTW))
    lin = _tr_kernel(tt, aux)
    table_lin = jnp.reshape(lin, (VOCAB_ROWS, EMBED))
    pooled = _pool_kernel(x, table_lin)

    wp = jnp.zeros((EMBED, LPAD), jnp.float32).at[:, :LABELS].set(W.T)
    bp = jnp.zeros((8, LPAD), jnp.float32).at[:, :LABELS].set(b[None, :])
    out = pl.pallas_call(
        _fc_body,
        grid=(BATCH // BM,),
        in_specs=[
            pl.BlockSpec((BM, EMBED), lambda i: (i, 0)),
            pl.BlockSpec((EMBED, LPAD), lambda i: (0, 0)),
            pl.BlockSpec((8, LPAD), lambda i: (0, 0)),
        ],
        out_specs=pl.BlockSpec((BM, LPAD), lambda i: (i, 0)),
        out_shape=jax.ShapeDtypeStruct((BATCH, LPAD), jnp.float32),
    )(pooled, wp, bp)
    return out[:, :LABELS]


# speedup vs baseline: 1.3182x; 1.3182x over previous
import functools

import jax
import jax.numpy as jnp
from jax import lax
from jax.experimental import pallas as pl
from jax.experimental.pallas import tpu as pltpu
from jax.experimental.pallas import tpu_sc as plsc

BATCH = 4096
MAXLEN = 200
VOCAB_ROWS = 1000000
EMBED = 64
LABELS = 100
LANES = 16
NC = 2
NS = 16
NW = NC * NS
B_PER_W = BATCH // NW
CHUNKS = ((0, 128), (128, 72))


NBUF = 3
TPAD = 2 * EMBED
UNROLL = 8
NCH = EMBED // LANES


def _pool_body(x_hbm, table_hbm, out_hbm, idx_v, rows_v, pooled_v, sems):
    w = lax.axis_index("s") * NC + lax.axis_index("c")
    base = w * B_PER_W
    pltpu.sync_copy(x_hbm.at[pl.ds(base, B_PER_W)], idx_v)

    def fire(b, k):
        for off, n in CHUNKS:
            pltpu.async_copy(
                table_hbm.at[idx_v.at[b, pl.ds(off, n)]],
                rows_v.at[k].at[pl.ds(off, n)],
                sems.at[k],
            )

    def drain(k):
        pltpu.make_async_copy(
            table_hbm.at[pl.ds(0, MAXLEN)], rows_v.at[k], sems.at[k]
        ).wait()

    def reduce_row(b, k):
        rv = rows_v.at[k]
        zero = jnp.zeros((LANES,), jnp.float32)

        def step(t, accs):
            accs = list(accs)
            for j in range(UNROLL):
                r = t * UNROLL + j
                for c in range(NCH):
                    a = c + NCH * (j % 2)
                    accs[a] = accs[a] + rv[r, pl.ds(c * LANES, LANES)]
            return tuple(accs)

        accs = lax.fori_loop(0, MAXLEN // UNROLL, step, (zero,) * (2 * NCH))
        scale = jnp.float32(1.0 / MAXLEN)
        for c in range(NCH):
            pooled_v[b, pl.ds(c * LANES, LANES)] = (accs[c] + accs[c + NCH]) * scale

    for k in range(NBUF):
        fire(k, k)

    def group(g, carry):
        for k in range(NBUF):
            b = g * NBUF + k

            @pl.when(b < B_PER_W)
            def _():
                drain(k)

                @pl.when(b + NBUF < B_PER_W)
                def _():
                    fire(b + NBUF, k)

                reduce_row(b, k)
        return carry

    lax.fori_loop(0, -(-B_PER_W // NBUF), group, 0)
    pltpu.sync_copy(pooled_v, out_hbm.at[pl.ds(base, B_PER_W)])


_pool_kernel = functools.partial(
    pl.kernel,
    out_type=jax.ShapeDtypeStruct((BATCH, EMBED), jnp.float32),
    mesh=plsc.VectorSubcoreMesh(core_axis_name="c", subcore_axis_name="s"),
    scratch_types=[
        pltpu.VMEM((B_PER_W, MAXLEN), jnp.int32),
        pltpu.VMEM((NBUF, MAXLEN, TPAD), jnp.float32),
        pltpu.VMEM((B_PER_W, EMBED), jnp.float32),
        pltpu.SemaphoreType.DMA((NBUF,)),
    ],
    compiler_params=pltpu.CompilerParams(use_tc_tiling_on_sc=False),
)(_pool_body)


TBLK = 256
OUTW = 128
NBLK = VOCAB_ROWS // TBLK
TAIL = VOCAB_ROWS - NBLK * TBLK
BPW = -(-NBLK // NW)


def _tr_body(tt_hbm, aux_hbm, out_hbm, in_v, out_v, aux_v, in_sem, out_sem):
    w = lax.axis_index("s") * NC + lax.axis_index("c")

    @pl.when(w == 0)
    def _():
        pltpu.sync_copy(aux_hbm, aux_v)
        pltpu.sync_copy(aux_v, out_hbm.at[pl.ds(NBLK * (TBLK // 2), TAIL // 2)])

    def blk(t):
        return w + NW * t

    def fire_in(t, p):
        pltpu.async_copy(
            tt_hbm.at[:, pl.ds(blk(t) * TBLK, TBLK)],
            in_v.at[p, :, pl.ds(0, TBLK)],
            in_sem.at[p],
        )

    def wait_in(p):
        pltpu.make_async_copy(
            tt_hbm.at[:, pl.ds(0, TBLK)],
            in_v.at[p, :, pl.ds(0, TBLK)],
            in_sem.at[p],
        ).wait()

    def fire_out(t, p):
        pltpu.async_copy(
            out_v.at[p], out_hbm.at[pl.ds(blk(t) * (TBLK // 2), TBLK // 2)],
            out_sem.at[p],
        )

    def wait_out(p):
        pltpu.make_async_copy(
            out_v.at[p], out_hbm.at[pl.ds(0, TBLK // 2)], out_sem.at[p]
        ).wait()

    @pl.when(blk(0) < NBLK)
    def _():
        fire_in(0, 0)

    lanes = lax.iota(jnp.int32, LANES)
    row_vecs = tuple(c * LANES + lanes for c in range(NCH))

    def step(g, carry):
        for k in range(2):
            t = 2 * g + k
            p = k

            @pl.when(blk(t) < NBLK)
            def _():
                wait_in(p)

                @pl.when(blk(t + 1) < NBLK)
                def _():
                    fire_in(t + 1, 1 - p)

                @pl.when(t >= 2)
                def _():
                    wait_out(p)

                in2 = in_v.at[p]
                out2 = out_v.at[p]

                @plsc.parallel_loop(
                    0, TBLK // 2, unroll=8,
                    carry=jnp.zeros((LANES,), jnp.int32),
                )
                def _(v2, col0):
                    for half in range(2):
                        col = col0 + half
                        for c in range(NCH):
                            val = plsc.load_gather(in2, [row_vecs[c], col])
                            out2[v2, pl.ds(half * EMBED + c * LANES, LANES)] = val
                    return col0 + 2

                fire_out(t, p)
        return carry

    lax.fori_loop(0, (BPW + 1) // 2, step, 0)
    wait_out(0)
    wait_out(1)


_tr_kernel = functools.partial(
    pl.kernel,
    out_type=jax.ShapeDtypeStruct((VOCAB_ROWS * EMBED // OUTW, OUTW), jnp.float32),
    mesh=plsc.VectorSubcoreMesh(core_axis_name="c", subcore_axis_name="s"),
    scratch_types=[
        pltpu.VMEM((2, EMBED, TBLK), jnp.float32),
        pltpu.VMEM((2, TBLK // 2, OUTW), jnp.float32),
        pltpu.VMEM((TAIL // 2, OUTW), jnp.float32),
        pltpu.SemaphoreType.DMA((2,)),
        pltpu.SemaphoreType.DMA((2,)),
    ],
    compiler_params=pltpu.CompilerParams(
        use_tc_tiling_on_sc=True, needs_layout_passes=False
    ),
)(_tr_body)


LPAD = 128
BM = 512


def _fc_body(p_ref, w_ref, b_ref, o_ref):
    o_ref[...] = (
        jnp.dot(p_ref[...], w_ref[...], preferred_element_type=jnp.float32)
        + b_ref[0:1, :]
    )


@jax.jit
def kernel(x, table, W, b):
    x = x.astype(jnp.int32)
    table_pad = jnp.pad(table, ((0, 0), (0, TPAD - EMBED)))
    pooled = _pool_kernel(x, table_pad)

    wp = jnp.zeros((EMBED, LPAD), jnp.float32).at[:, :LABELS].set(W.T)
    bp = jnp.zeros((8, LPAD), jnp.float32).at[:, :LABELS].set(b[None, :])
    out = pl.pallas_call(
        _fc_body,
        grid=(BATCH // BM,),
        in_specs=[
            pl.BlockSpec((BM, EMBED), lambda i: (i, 0)),
            pl.BlockSpec((EMBED, LPAD), lambda i: (0, 0)),
            pl.BlockSpec((8, LPAD), lambda i: (0, 0)),
        ],
        out_specs=pl.BlockSpec((BM, LPAD), lambda i: (i, 0)),
        out_shape=jax.ShapeDtypeStruct((BATCH, LPAD), jnp.float32),
    )(pooled, wp, bp)
    return out[:, :LABELS]


# speedup vs baseline: 1.7564x; 1.3325x over previous
import functools

import jax
import jax.numpy as jnp
from jax import lax
from jax.experimental import pallas as pl
from jax.experimental.pallas import tpu as pltpu
from jax.experimental.pallas import tpu_sc as plsc

BATCH = 4096
MAXLEN = 200
VOCAB_ROWS = 1000000
EMBED = 64
LABELS = 100
LANES = 16
NC = 2
NS = 16
NW = NC * NS
B_PER_W = BATCH // NW
CHUNKS = ((0, 128), (128, 72))

NBUF = 4
UNROLL = 8
NCH = EMBED // LANES


def _pool_body(x_hbm, table_hbm, out_hbm, idx_v, rows_v, pooled_v, sems):
    w = lax.axis_index("s") * NC + lax.axis_index("c")
    base = w * B_PER_W
    pltpu.sync_copy(x_hbm.at[pl.ds(base, B_PER_W)], idx_v)

    def fire(b, k):
        for off, n in CHUNKS:
            pltpu.async_copy(
                table_hbm.at[idx_v.at[b, pl.ds(off, n)]],
                rows_v.at[k].at[pl.ds(off, n)],
                sems.at[k],
            )

    def drain(k):
        pltpu.make_async_copy(
            table_hbm.at[pl.ds(0, MAXLEN)], rows_v.at[k], sems.at[k]
        ).wait()

    def reduce_row(b, k):
        rv = rows_v.at[k]
        zero = jnp.zeros((LANES,), jnp.float32)

        def step(t, accs):
            accs = list(accs)
            for j in range(UNROLL):
                r = t * UNROLL + j
                for c in range(NCH):
                    a = c + NCH * (j % 2)
                    accs[a] = accs[a] + rv[r, pl.ds(c * LANES, LANES)]
            return tuple(accs)

        accs = lax.fori_loop(0, MAXLEN // UNROLL, step, (zero,) * (2 * NCH))
        scale = jnp.float32(1.0 / MAXLEN)
        for c in range(NCH):
            pooled_v[b, pl.ds(c * LANES, LANES)] = (accs[c] + accs[c + NCH]) * scale

    for k in range(NBUF):
        fire(k, k)

    def group(g, carry):
        for k in range(NBUF):
            b = g * NBUF + k
            drain(k)

            @pl.when(g < B_PER_W // NBUF - 1)
            def _():
                fire(b + NBUF, k)

            reduce_row(b, k)
        return carry

    lax.fori_loop(0, B_PER_W // NBUF, group, 0)
    pltpu.sync_copy(pooled_v, out_hbm.at[pl.ds(base, B_PER_W)])


_pool_kernel = functools.partial(
    pl.kernel,
    out_type=jax.ShapeDtypeStruct((BATCH, EMBED), jnp.float32),
    mesh=plsc.VectorSubcoreMesh(core_axis_name="c", subcore_axis_name="s"),
    scratch_types=[
        pltpu.VMEM((B_PER_W, MAXLEN), jnp.int32),
        pltpu.VMEM((NBUF, MAXLEN, EMBED), jnp.float32),
        pltpu.VMEM((B_PER_W, EMBED), jnp.float32),
        pltpu.SemaphoreType.DMA((NBUF,)),
    ],
    compiler_params=pltpu.CompilerParams(use_tc_tiling_on_sc=False),
)(_pool_body)


LPAD = 128
BM = 512


def _fc_body(p_ref, w_ref, b_ref, o_ref):
    o_ref[...] = (
        jnp.dot(p_ref[...], w_ref[...], preferred_element_type=jnp.float32)
        + b_ref[0:1, :]
    )


@jax.jit
def kernel(x, table, W, b):
    x = x.astype(jnp.int32)
    pooled = _pool_kernel(x, table)

    wp = jnp.zeros((EMBED, LPAD), jnp.float32).at[:, :LABELS].set(W.T)
    bp = jnp.zeros((8, LPAD), jnp.float32).at[:, :LABELS].set(b[None, :])
    out = pl.pallas_call(
        _fc_body,
        grid=(BATCH // BM,),
        in_specs=[
            pl.BlockSpec((BM, EMBED), lambda i: (i, 0)),
            pl.BlockSpec((EMBED, LPAD), lambda i: (0, 0)),
            pl.BlockSpec((8, LPAD), lambda i: (0, 0)),
        ],
        out_specs=pl.BlockSpec((BM, LPAD), lambda i: (i, 0)),
        out_shape=jax.ShapeDtypeStruct((BATCH, LPAD), jnp.float32),
    )(pooled, wp, bp)
    return out[:, :LABELS]
